# baseline (speedup 1.0000x reference)
import functools

import jax
import jax.numpy as jnp
from jax import lax
from jax.experimental import pallas as pl
from jax.experimental.pallas import tpu as pltpu

try:
    from jax.experimental.pallas import tpu_sc as plsc
    _HAS_SC = True
except ImportError:
    _HAS_SC = False

N_USER = 10000
N_REV = 10000
DEG = 32
D = 128
H = 128

_NW = 32
_ROWS = (1000, 1000, 2000, 2000, 2000, 2000)
_STARTS = (0, 1000, 2000, 4000, 6000, 8000)
_P = len(_ROWS)
_NCHUNK = 25
_GW = 2 * D


def _gather_sc(phase, rows, idx4, table):
    mesh = plsc.VectorSubcoreMesh(core_axis_name="c", subcore_axis_name="s")

    chunk = rows // _NCHUNK
    nbuf = 4
    nround = _NCHUNK // nbuf
    last = _NCHUNK - 1

    @functools.partial(
        pl.kernel,
        mesh=mesh,
        out_type=jax.ShapeDtypeStruct((rows * DEG, _GW), jnp.int32),
        scratch_types=[
            pltpu.VMEM((_NCHUNK, chunk), jnp.int32),
        ] + [pltpu.VMEM((chunk, _GW), jnp.int32)] * nbuf
          + [pltpu.SemaphoreType.DMA] * (2 * nbuf),
    )
    def gather_kernel(idx_hbm, table_hbm, out_hbm, idx_v, *bufsem):
        bufs = bufsem[:nbuf]
        gsem = bufsem[nbuf:2 * nbuf]
        wsem = bufsem[2 * nbuf:]
        wid = lax.axis_index("s") * 2 + lax.axis_index("c")
        base = wid * rows
        pltpu.sync_copy(idx_hbm.at[phase, wid], idx_v)

        def g(j, s):
            src = table_hbm.at[idx_v.at[j]]
            pltpu.make_async_copy(src, bufs[s], gsem[s]).start()

        def gwait(s):
            src = table_hbm.at[idx_v.at[0]]
            pltpu.make_async_copy(src, bufs[s], gsem[s]).wait()

        def wb(j, s):
            dst = out_hbm.at[pl.ds(base + j * chunk, chunk)]
            pltpu.make_async_copy(bufs[s], dst, wsem[s]).start()

        def wbwait(s):
            dst = out_hbm.at[pl.ds(base, chunk)]
            pltpu.make_async_copy(bufs[s], dst, wsem[s]).wait()

        for s in range(nbuf):
            g(s, s)

        def body(jj, carry):
            j0 = jj * nbuf
            for s in range(nbuf):
                j = j0 + s
                gwait(s)
                wb(j, s)

                @pl.when(j + nbuf <= last)
                def _(j=j, s=s):
                    wbwait(s)
                    g(j + nbuf, s)
            return carry

        lax.fori_loop(0, nround, body, 0)
        gwait(0)
        wb(last, 0)
        for s in range(nbuf):
            wbwait(s)

    return gather_kernel(idx4, table)


_BU = 2000


def _rne_bf16_bits(v):
    bits = lax.bitcast_convert_type(v, jnp.int32)
    rnd = jnp.int32(0x7FFF) + (lax.shift_right_logical(bits, 16) & 1)
    return lax.shift_right_logical(bits + rnd, 16)


def _stage_u_body(x_ref, w_ref, b_ref, u_ref):
    xb = x_ref[...].astype(jnp.bfloat16)
    u = jnp.dot(xb, w_ref[...], preferred_element_type=jnp.float32) + b_ref[...]
    lo = _rne_bf16_bits(u[:, 0:2 * D])
    hi = _rne_bf16_bits(u[:, 2 * D:4 * D])
    u_ref[...] = lo | lax.shift_left(hi, 16)


def _stage_u(x_user, wihT, b):
    full = lambda i: (0, 0)
    return pl.pallas_call(
        _stage_u_body,
        grid=(N_USER // _BU,),
        in_specs=[
            pl.BlockSpec((_BU, D), lambda i: (i, 0)),
            pl.BlockSpec((D, 4 * D), full),
            pl.BlockSpec((1, 4 * D), full),
        ],
        out_specs=pl.BlockSpec((_BU, _GW), lambda i: (i, 0)),
        out_shape=jax.ShapeDtypeStruct((N_USER, _GW), jnp.int32),
    )(x_user, wihT, b)


def _make_stage_a_body(br):
    return functools.partial(_stage_a_body, br)


def _stage_a_body(br, gx_ref, xr_ref, whhT_ref, wself_ref, wneigh_ref,
                  bsage_ref, lng_ref, lnb_ref, hn_ref, stats_ref):
    f32 = jnp.float32
    hb2 = br // 2
    z = jnp.zeros((hb2, H), f32)

    def half(gxw, gh, c):
        lo = lax.bitcast_convert_type(lax.shift_left(gxw, 16), f32)
        hi = lax.bitcast_convert_type(gxw & jnp.int32(-65536), f32)
        ti = jnp.tanh(lo[:, 0:H] + gh[:, 0:H])
        tf = jnp.tanh(lo[:, H:2 * H] + gh[:, H:2 * H])
        tg = jnp.tanh(hi[:, 0:H] + gh[:, 2 * H:3 * H])
        to = jnp.tanh(hi[:, H:2 * H] + gh[:, 3 * H:4 * H])
        c = 0.5 * ((tf * c + ti * tg) + (c + tg))
        tc = jnp.tanh(c)
        h = 0.5 * (to * tc + tc)
        return h, c

    def substep(t, h1, c1, h2, c2):
        gh1 = jnp.dot(h1.astype(jnp.bfloat16), whhT_ref[...],
                      preferred_element_type=f32)
        gh2 = jnp.dot(h2.astype(jnp.bfloat16), whhT_ref[...],
                      preferred_element_type=f32)
        gxw = gx_ref[t]
        h1, c1 = half(gxw[0:hb2], gh1, c1)
        h2, c2 = half(gxw[hb2:br], gh2, c2)
        return h1, c1, h2, c2

    def step(u, hc):
        h1, c1, h2, c2 = hc
        for k in range(8):
            h1, c1, h2, c2 = substep(8 * u + k, h1, c1, h2, c2)
        return (h1, c1, h2, c2)

    h1, _, h2, _ = lax.fori_loop(0, DEG // 8, step, (z, z, z, z))
    h = jnp.concatenate([h1, h2], axis=0)

    rst = (jnp.dot(xr_ref[...], wself_ref[...], preferred_element_type=f32)
           + jnp.dot(h, wneigh_ref[...], preferred_element_type=f32)
           + bsage_ref[...])
    rst = jax.nn.gelu(rst)
    mu = jnp.mean(rst, axis=1, keepdims=True)
    var = jnp.mean((rst - mu) ** 2, axis=1, keepdims=True)
    hn = (rst - mu) * lax.rsqrt(var + 1e-5) * lng_ref[...] + lnb_ref[...]
    hn_ref[...] = hn

    @pl.when(pl.program_id(0) == 0)
    def _():
        stats_ref[...] = jnp.zeros_like(stats_ref)

    s1 = jnp.sum(hn, axis=0, keepdims=True)
    s2 = jnp.sum(hn * hn, axis=0, keepdims=True)
    stats_ref[...] += jnp.concatenate([s1, s2], axis=0)


_full = lambda i: (0, 0)


def _stage_a_phase(start, rows, gx3, x_review, whhT, wself, wneigh, bsage,
                   lng, lnb):
    br = rows // 5
    boff = start // br
    return pl.pallas_call(
        _make_stage_a_body(br),
        grid=(5,),
        in_specs=[
            pl.BlockSpec((DEG, br, _GW), lambda i: (0, i, 0)),
            pl.BlockSpec((br, D), lambda i, _b=boff: (i + _b, 0)),
            pl.BlockSpec((H, 4 * D), _full),
            pl.BlockSpec((D, H), _full),
            pl.BlockSpec((D, H), _full),
            pl.BlockSpec((1, H), _full),
            pl.BlockSpec((1, H), _full),
            pl.BlockSpec((1, H), _full),
        ],
        out_specs=[
            pl.BlockSpec((br, H), lambda i: (i, 0)),
            pl.BlockSpec((2, H), _full),
        ],
        out_shape=[
            jax.ShapeDtypeStruct((rows, H), jnp.float32),
            jax.ShapeDtypeStruct((2, H), jnp.float32),
        ],
    )(gx3, x_review, whhT, wself, wneigh, bsage, lng, lnb)


_BB = 200
_BSTART = tuple(s // _BB for s in _STARTS)
_BNBLK = tuple(r // _BB for r in _ROWS)


def _stage_b_body(*refs):
    hn_refs = refs[:_P]
    st_refs = refs[_P:2 * _P]
    bng_ref, bnb_ref, w1_ref, b1_ref, w2_ref, b2_ref, out_ref = refs[2 * _P:]
    f32 = jnp.float32
    s = st_refs[0][...]
    for r in st_refs[1:]:
        s = s + r[...]
    bm = s[0:1, :] * (1.0 / N_REV)
    bv = s[1:2, :] * (1.0 / N_REV) - bm * bm
    scale = lax.rsqrt(bv + 1e-5) * bng_ref[...]
    i = pl.program_id(0)
    hn = hn_refs[0][...]
    for q in range(1, _P):
        hn = jnp.where(i >= _BSTART[q], hn_refs[q][...], hn)
    hb = (hn - bm) * scale + bnb_ref[...]
    t1 = jax.nn.gelu(jnp.dot(hb, w1_ref[...], preferred_element_type=f32)
                     + b1_ref[...])
    out_ref[...] = jnp.dot(t1, w2_ref[...], preferred_element_type=f32) + b2_ref[...]


def _stage_b(hns, stats_parts, bng, bnb, w1, b1, w2, b2):
    full = lambda i: (0, 0)

    def hn_map(q):
        return lambda i: (jnp.clip(i - _BSTART[q], 0, _BNBLK[q] - 1), 0)

    return pl.pallas_call(
        _stage_b_body,
        grid=(N_REV // _BB,),
        in_specs=[pl.BlockSpec((_BB, H), hn_map(q)) for q in range(_P)]
        + [pl.BlockSpec((2, H), full)] * _P
        + [
            pl.BlockSpec((1, H), full),
            pl.BlockSpec((1, H), full),
            pl.BlockSpec((H, H), full),
            pl.BlockSpec((1, H), full),
            pl.BlockSpec((H, 2), full),
            pl.BlockSpec((1, 2), full),
        ],
        out_specs=pl.BlockSpec((_BB, 2), lambda i: (i, 0)),
        out_shape=jax.ShapeDtypeStruct((N_REV, 2), jnp.float32),
    )(*hns, *stats_parts, bng, bnb, w1, b1, w2, b2)


def kernel(x_user, x_review, edge_src, W_ih, W_hh, b_ih, b_hh, W_self,
           W_neigh, b_sage, ln_g, ln_b, bn_g, bn_b, W1, b1, W2, b2):
    er = edge_src.reshape(N_REV, DEG)
    idx_small = jnp.transpose(er[0:2000].reshape(2, 1000, DEG),
                              (0, 2, 1)).reshape(2, _NW, _NCHUNK, 40)
    idx_big = jnp.transpose(er[2000:].reshape(4, 2000, DEG),
                            (0, 2, 1)).reshape(4, _NW, _NCHUNK, 80)

    gate_scale = jnp.concatenate(
        [jnp.full((D,), 0.5, jnp.float32), jnp.full((D,), 0.5, jnp.float32),
         jnp.ones((D,), jnp.float32), jnp.full((D,), 0.5, jnp.float32)])
    wihT = (W_ih.T * gate_scale).astype(jnp.bfloat16)
    whhT = (W_hh.T * gate_scale).astype(jnp.bfloat16)
    b = ((b_ih + b_hh) * gate_scale).reshape(1, 4 * D)

    table = _stage_u(x_user, wihT, b)
    gx = [_gather_sc(p if p < 2 else p - 2, _ROWS[p],
                     idx_small if p < 2 else idx_big, table).reshape(
              DEG, _ROWS[p], _GW) for p in range(_P)]

    hns, stats_parts = [], []
    for p in range(_P):
        hn_p, st_p = _stage_a_phase(_STARTS[p], _ROWS[p], gx[p], x_review,
                                    whhT, W_self, W_neigh,
                                    b_sage.reshape(1, H),
                                    ln_g.reshape(1, H), ln_b.reshape(1, H))
        hns.append(hn_p)
        stats_parts.append(st_p)

    return _stage_b(hns, stats_parts, bn_g.reshape(1, H), bn_b.reshape(1, H),
                    W1, b1.reshape(1, H), W2, b2.reshape(1, 2))


# speedup vs baseline: 1.1126x; 1.1126x over previous
import functools

import jax
import jax.numpy as jnp
from jax import lax
from jax.experimental import pallas as pl
from jax.experimental.pallas import tpu as pltpu

try:
    from jax.experimental.pallas import tpu_sc as plsc
    _HAS_SC = True
except ImportError:
    _HAS_SC = False

N_USER = 10000
N_REV = 10000
DEG = 32
D = 128
H = 128

_NW = 32
_P = 5
_PR = N_REV // _P
_B_PHASE = _PR * DEG
_B_PER_W = _B_PHASE // _NW
_CHUNK = 80
_NCHUNK = _B_PER_W // _CHUNK
_NPAIR = (_NCHUNK - 1) // 2
_GW = 2 * D


def _gather_sc(phase, idx, table):
    mesh = plsc.VectorSubcoreMesh(core_axis_name="c", subcore_axis_name="s")

    nbuf = 4
    nround = _NCHUNK // nbuf
    last = _NCHUNK - 1

    @functools.partial(
        pl.kernel,
        mesh=mesh,
        out_type=jax.ShapeDtypeStruct((_B_PHASE, _GW), jnp.int32),
        scratch_types=[
            pltpu.VMEM((_NCHUNK, _CHUNK), jnp.int32),
        ] + [pltpu.VMEM((_CHUNK, _GW), jnp.int32)] * nbuf
          + [pltpu.SemaphoreType.DMA] * (2 * nbuf),
    )
    def gather_kernel(idx_hbm, table_hbm, out_hbm, idx_v, *bufsem):
        bufs = bufsem[:nbuf]
        gsem = bufsem[nbuf:2 * nbuf]
        wsem = bufsem[2 * nbuf:]
        wid = lax.axis_index("s") * 2 + lax.axis_index("c")
        base = wid * _B_PER_W
        pltpu.sync_copy(idx_hbm.at[phase, wid], idx_v)

        def g(j, s):
            pltpu.make_async_copy(table_hbm.at[idx_v.at[j]], bufs[s],
                                  gsem[s]).start()

        def gwait(s):
            pltpu.make_async_copy(table_hbm.at[idx_v.at[0]], bufs[s],
                                  gsem[s]).wait()

        def wb(j, s):
            dst = out_hbm.at[pl.ds(base + j * _CHUNK, _CHUNK)]
            pltpu.make_async_copy(bufs[s], dst, wsem[s]).start()

        def wbwait(s):
            dst = out_hbm.at[pl.ds(base, _CHUNK)]
            pltpu.make_async_copy(bufs[s], dst, wsem[s]).wait()

        for s in range(nbuf):
            g(s, s)

        def body(jj, carry):
            j0 = jj * nbuf
            for s in range(nbuf):
                j = j0 + s
                gwait(s)
                wb(j, s)

                @pl.when(j + nbuf <= last)
                def _(j=j, s=s):
                    wbwait(s)
                    g(j + nbuf, s)
            return carry

        lax.fori_loop(0, nround, body, 0)
        gwait(0)
        wb(last, 0)
        for s in range(nbuf):
            wbwait(s)

    return gather_kernel(idx, table)


_BU = 2000


def _rne_bf16_bits(v):
    bits = lax.bitcast_convert_type(v, jnp.int32)
    rnd = jnp.int32(0x7FFF) + (lax.shift_right_logical(bits, 16) & 1)
    return lax.shift_right_logical(bits + rnd, 16)


def _stage_u_body(x_ref, w_ref, b_ref, u_ref):
    xb = x_ref[...].astype(jnp.bfloat16)
    u = jnp.dot(xb, w_ref[...], preferred_element_type=jnp.float32) + b_ref[...]
    lo = _rne_bf16_bits(u[:, 0:2 * D])
    hi = _rne_bf16_bits(u[:, 2 * D:4 * D])
    u_ref[...] = lo | lax.shift_left(hi, 16)


def _stage_u(x_user, wihT, b):
    full = lambda i: (0, 0)
    return pl.pallas_call(
        _stage_u_body,
        grid=(N_USER // _BU,),
        in_specs=[
            pl.BlockSpec((_BU, D), lambda i: (i, 0)),
            pl.BlockSpec((D, 4 * D), full),
            pl.BlockSpec((1, 4 * D), full),
        ],
        out_specs=pl.BlockSpec((_BU, _GW), lambda i: (i, 0)),
        out_shape=jax.ShapeDtypeStruct((N_USER, _GW), jnp.int32),
    )(x_user, wihT, b)


_BR = 400
_NBLK = N_REV // _BR


_HB = _BR // 2


def _stage_a_body(gx_ref, xr_ref, whhT_ref, wself_ref, wneigh_ref,
                  bsage_ref, lng_ref, lnb_ref, hn_ref, stats_ref):
    f32 = jnp.float32
    z = jnp.zeros((_HB, H), f32)

    def half(gxw, gh, c):
        lo = lax.bitcast_convert_type(lax.shift_left(gxw, 16), f32)
        hi = lax.bitcast_convert_type(gxw & jnp.int32(-65536), f32)
        ti = jnp.tanh(lo[:, 0:H] + gh[:, 0:H])
        tf = jnp.tanh(lo[:, H:2 * H] + gh[:, H:2 * H])
        tg = jnp.tanh(hi[:, 0:H] + gh[:, 2 * H:3 * H])
        to = jnp.tanh(hi[:, H:2 * H] + gh[:, 3 * H:4 * H])
        c = 0.5 * ((tf * c + ti * tg) + (c + tg))
        tc = jnp.tanh(c)
        h = 0.5 * (to * tc + tc)
        return h, c

    def substep(t, h1, c1, h2, c2):
        gh1 = jnp.dot(h1.astype(jnp.bfloat16), whhT_ref[...],
                      preferred_element_type=f32)
        gh2 = jnp.dot(h2.astype(jnp.bfloat16), whhT_ref[...],
                      preferred_element_type=f32)
        gxw = gx_ref[t]
        h1, c1 = half(gxw[0:_HB], gh1, c1)
        h2, c2 = half(gxw[_HB:_BR], gh2, c2)
        return h1, c1, h2, c2

    def step(u, hc):
        h1, c1, h2, c2 = hc
        for k in range(8):
            h1, c1, h2, c2 = substep(8 * u + k, h1, c1, h2, c2)
        return (h1, c1, h2, c2)

    h1, _, h2, _ = lax.fori_loop(0, DEG // 8, step, (z, z, z, z))
    h = jnp.concatenate([h1, h2], axis=0)

    rst = (jnp.dot(xr_ref[...], wself_ref[...], preferred_element_type=f32)
           + jnp.dot(h, wneigh_ref[...], preferred_element_type=f32)
           + bsage_ref[...])
    rst = jax.nn.gelu(rst)
    mu = jnp.mean(rst, axis=1, keepdims=True)
    var = jnp.mean((rst - mu) ** 2, axis=1, keepdims=True)
    hn = (rst - mu) * lax.rsqrt(var + 1e-5) * lng_ref[...] + lnb_ref[...]
    hn_ref[...] = hn

    @pl.when(pl.program_id(0) == 0)
    def _():
        stats_ref[...] = jnp.zeros_like(stats_ref)

    s1 = jnp.sum(hn, axis=0, keepdims=True)
    s2 = jnp.sum(hn * hn, axis=0, keepdims=True)
    stats_ref[...] += jnp.concatenate([s1, s2], axis=0)


_full = lambda i: (0, 0)


def _stage_a_phase(p, gx3, x_review, whhT, wself, wneigh, bsage, lng, lnb):
    nblk = _PR // _BR
    return pl.pallas_call(
        _stage_a_body,
        grid=(nblk,),
        in_specs=[
            pl.BlockSpec((DEG, _BR, _GW), lambda i: (0, i, 0)),
            pl.BlockSpec((_BR, D), lambda i, _p=p * (_PR // _BR): (i + _p, 0)),
            pl.BlockSpec((H, 4 * D), _full),
            pl.BlockSpec((D, H), _full),
            pl.BlockSpec((D, H), _full),
            pl.BlockSpec((1, H), _full),
            pl.BlockSpec((1, H), _full),
            pl.BlockSpec((1, H), _full),
        ],
        out_specs=[
            pl.BlockSpec((_BR, H), lambda i: (i, 0)),
            pl.BlockSpec((2, H), _full),
        ],
        out_shape=[
            jax.ShapeDtypeStruct((_PR, H), jnp.float32),
            jax.ShapeDtypeStruct((2, H), jnp.float32),
        ],
    )(gx3, x_review, whhT, wself, wneigh, bsage, lng, lnb)


_BPP = _PR // _BR


def _stage_b_body(*refs):
    hn_refs = refs[:_P]
    st_refs = refs[_P:2 * _P]
    bng_ref, bnb_ref, w1_ref, b1_ref, w2_ref, b2_ref, out_ref = refs[2 * _P:]
    f32 = jnp.float32
    s = st_refs[0][...]
    for r in st_refs[1:]:
        s = s + r[...]
    bm = s[0:1, :] * (1.0 / N_REV)
    bv = s[1:2, :] * (1.0 / N_REV) - bm * bm
    scale = lax.rsqrt(bv + 1e-5) * bng_ref[...]
    p = pl.program_id(0) // _BPP
    hn = hn_refs[0][...]
    for q in range(1, _P):
        hn = jnp.where(p == q, hn_refs[q][...], hn)
    hb = (hn - bm) * scale + bnb_ref[...]
    t1 = jax.nn.gelu(jnp.dot(hb, w1_ref[...], preferred_element_type=f32)
                     + b1_ref[...])
    out_ref[...] = jnp.dot(t1, w2_ref[...], preferred_element_type=f32) + b2_ref[...]


def _stage_b(hns, stats_parts, bng, bnb, w1, b1, w2, b2):
    full = lambda i: (0, 0)

    def hn_map(q):
        return lambda i: (jnp.clip(i - q * _BPP, 0, _BPP - 1), 0)

    return pl.pallas_call(
        _stage_b_body,
        grid=(N_REV // _BR,),
        in_specs=[pl.BlockSpec((_BR, H), hn_map(q)) for q in range(_P)]
        + [pl.BlockSpec((2, H), full)] * _P
        + [
            pl.BlockSpec((1, H), full),
            pl.BlockSpec((1, H), full),
            pl.BlockSpec((H, H), full),
            pl.BlockSpec((1, H), full),
            pl.BlockSpec((H, 2), full),
            pl.BlockSpec((1, 2), full),
        ],
        out_specs=pl.BlockSpec((_BR, 2), lambda i: (i, 0)),
        out_shape=jax.ShapeDtypeStruct((N_REV, 2), jnp.float32),
    )(*hns, *stats_parts, bng, bnb, w1, b1, w2, b2)


def kernel(x_user, x_review, edge_src, W_ih, W_hh, b_ih, b_hh, W_self,
           W_neigh, b_sage, ln_g, ln_b, bn_g, bn_b, W1, b1, W2, b2):
    idx_t = jnp.transpose(
        edge_src.reshape(_P, _PR, DEG), (0, 2, 1)
    ).reshape(_P, _NW, _NCHUNK, _CHUNK)

    gate_scale = jnp.concatenate(
        [jnp.full((D,), 0.5, jnp.float32), jnp.full((D,), 0.5, jnp.float32),
         jnp.ones((D,), jnp.float32), jnp.full((D,), 0.5, jnp.float32)])
    wihT = (W_ih.T * gate_scale).astype(jnp.bfloat16)
    whhT = (W_hh.T * gate_scale).astype(jnp.bfloat16)
    b = ((b_ih + b_hh) * gate_scale).reshape(1, 4 * D)

    table = _stage_u(x_user, wihT, b)
    gx = [_gather_sc(p, idx_t, table).reshape(DEG, _PR, _GW)
          for p in range(_P)]

    hns, stats_parts = [], []
    for p in range(_P):
        hn_p, st_p = _stage_a_phase(p, gx[p], x_review, whhT, W_self,
                                    W_neigh, b_sage.reshape(1, H),
                                    ln_g.reshape(1, H), ln_b.reshape(1, H))
        hns.append(hn_p)
        stats_parts.append(st_p)

    return _stage_b(hns, stats_parts, bn_g.reshape(1, H), bn_b.reshape(1, H),
                    W1, b1.reshape(1, H), W2, b2.reshape(1, 2))


# speedup vs baseline: 1.1127x; 1.0001x over previous
import functools

import jax
import jax.numpy as jnp
from jax import lax
from jax.experimental import pallas as pl
from jax.experimental.pallas import tpu as pltpu

from jax.experimental.pallas import tpu_sc as plsc

N_USER = 10000
N_REV = 10000
DEG = 32
D = 128
H = 128

_NW = 32
_P = 5
_PR = N_REV // _P
_B_PHASE = _PR * DEG
_B_PER_W = _B_PHASE // _NW
_CHUNK = 80
_NCHUNK = _B_PER_W // _CHUNK
_NPAIR = (_NCHUNK - 1) // 2
_GW = 2 * D


def _gather_sc(phase, idx, table):
    mesh = plsc.VectorSubcoreMesh(core_axis_name="c", subcore_axis_name="s")

    nbuf = 4
    nround = _NCHUNK // nbuf
    last = _NCHUNK - 1

    @functools.partial(
        pl.kernel,
        mesh=mesh,
        out_type=jax.ShapeDtypeStruct((_B_PHASE, _GW), jnp.int32),
        scratch_types=[
            pltpu.VMEM((_NCHUNK, _CHUNK), jnp.int32),
        ] + [pltpu.VMEM((_CHUNK, _GW), jnp.int32)] * nbuf
          + [pltpu.SemaphoreType.DMA] * (2 * nbuf),
    )
    def gather_kernel(idx_hbm, table_hbm, out_hbm, idx_v, *bufsem):
        bufs = bufsem[:nbuf]
        gsem = bufsem[nbuf:2 * nbuf]
        wsem = bufsem[2 * nbuf:]
        wid = lax.axis_index("s") * 2 + lax.axis_index("c")
        base = wid * _B_PER_W
        pltpu.sync_copy(idx_hbm.at[phase, wid], idx_v)

        def g(j, s):
            pltpu.make_async_copy(table_hbm.at[idx_v.at[j]], bufs[s],
                                  gsem[s]).start()

        def gwait(s):
            pltpu.make_async_copy(table_hbm.at[idx_v.at[0]], bufs[s],
                                  gsem[s]).wait()

        def wb(j, s):
            dst = out_hbm.at[pl.ds(base + j * _CHUNK, _CHUNK)]
            pltpu.make_async_copy(bufs[s], dst, wsem[s]).start()

        def wbwait(s):
            dst = out_hbm.at[pl.ds(base, _CHUNK)]
            pltpu.make_async_copy(bufs[s], dst, wsem[s]).wait()

        for s in range(nbuf):
            g(s, s)

        def body(jj, carry):
            j0 = jj * nbuf
            for s in range(nbuf):
                j = j0 + s
                gwait(s)
                wb(j, s)

                @pl.when(j + nbuf <= last)
                def _(j=j, s=s):
                    wbwait(s)
                    g(j + nbuf, s)
            return carry

        lax.fori_loop(0, nround, body, 0)
        gwait(0)
        wb(last, 0)
        for s in range(nbuf):
            wbwait(s)

    return gather_kernel(idx, table)


_BU = 2000


def _rne_bf16_bits(v):
    bits = lax.bitcast_convert_type(v, jnp.int32)
    rnd = jnp.int32(0x7FFF) + (lax.shift_right_logical(bits, 16) & 1)
    return lax.shift_right_logical(bits + rnd, 16)


def _stage_u_body(x_ref, w_ref, b_ref, u_ref):
    xb = x_ref[...].astype(jnp.bfloat16)
    u = jnp.dot(xb, w_ref[...], preferred_element_type=jnp.float32) + b_ref[...]
    lo = _rne_bf16_bits(u[:, 0:2 * D])
    hi = _rne_bf16_bits(u[:, 2 * D:4 * D])
    u_ref[...] = lo | lax.shift_left(hi, 16)


def _stage_u(x_user, wihT, b):
    full = lambda i: (0, 0)
    return pl.pallas_call(
        _stage_u_body,
        grid=(N_USER // _BU,),
        in_specs=[
            pl.BlockSpec((_BU, D), lambda i: (i, 0)),
            pl.BlockSpec((D, 4 * D), full),
            pl.BlockSpec((1, 4 * D), full),
        ],
        out_specs=pl.BlockSpec((_BU, _GW), lambda i: (i, 0)),
        out_shape=jax.ShapeDtypeStruct((N_USER, _GW), jnp.int32),
    )(x_user, wihT, b)


_BR = 400
_NBLK = N_REV // _BR


_HB = _BR // 2


def _stage_a_body(gx_ref, xr_ref, whhT_ref, wself_ref, wneigh_ref,
                  bsage_ref, lng_ref, lnb_ref, hn_ref, stats_ref):
    f32 = jnp.float32
    z = jnp.zeros((_HB, H), f32)

    def half(gxw, gh, c):
        lo = lax.bitcast_convert_type(lax.shift_left(gxw, 16), f32)
        hi = lax.bitcast_convert_type(gxw & jnp.int32(-65536), f32)
        ti = jnp.tanh(lo[:, 0:H] + gh[:, 0:H])
        tf = jnp.tanh(lo[:, H:2 * H] + gh[:, H:2 * H])
        tg = jnp.tanh(hi[:, 0:H] + gh[:, 2 * H:3 * H])
        to = jnp.tanh(hi[:, H:2 * H] + gh[:, 3 * H:4 * H])
        c = 0.5 * ((tf * c + ti * tg) + (c + tg))
        tc = jnp.tanh(c)
        h = 0.5 * (to * tc + tc)
        return h, c

    def substep(t, h1, c1, h2, c2):
        gh1 = jnp.dot(h1.astype(jnp.bfloat16), whhT_ref[...],
                      preferred_element_type=f32)
        gh2 = jnp.dot(h2.astype(jnp.bfloat16), whhT_ref[...],
                      preferred_element_type=f32)
        gxw = gx_ref[t]
        h1, c1 = half(gxw[0:_HB], gh1, c1)
        h2, c2 = half(gxw[_HB:_BR], gh2, c2)
        return h1, c1, h2, c2

    def step(u, hc):
        h1, c1, h2, c2 = hc
        for k in range(8):
            h1, c1, h2, c2 = substep(8 * u + k, h1, c1, h2, c2)
        return (h1, c1, h2, c2)

    h1, _, h2, _ = lax.fori_loop(0, DEG // 8, step, (z, z, z, z))
    h = jnp.concatenate([h1, h2], axis=0)

    rst = (jnp.dot(xr_ref[...], wself_ref[...], preferred_element_type=f32)
           + jnp.dot(h, wneigh_ref[...], preferred_element_type=f32)
           + bsage_ref[...])
    rst = jax.nn.gelu(rst)
    mu = jnp.mean(rst, axis=1, keepdims=True)
    var = jnp.mean((rst - mu) ** 2, axis=1, keepdims=True)
    hn = (rst - mu) * lax.rsqrt(var + 1e-5) * lng_ref[...] + lnb_ref[...]
    hn_ref[...] = hn

    @pl.when(pl.program_id(0) == 0)
    def _():
        stats_ref[...] = jnp.zeros_like(stats_ref)

    s1 = jnp.sum(hn, axis=0, keepdims=True)
    s2 = jnp.sum(hn * hn, axis=0, keepdims=True)
    stats_ref[...] += jnp.concatenate([s1, s2], axis=0)


_full = lambda i: (0, 0)


def _stage_a_phase(p, gx3, x_review, whhT, wself, wneigh, bsage, lng, lnb):
    nblk = _PR // _BR
    return pl.pallas_call(
        _stage_a_body,
        grid=(nblk,),
        in_specs=[
            pl.BlockSpec((DEG, _BR, _GW), lambda i: (0, i, 0)),
            pl.BlockSpec((_BR, D), lambda i, _p=p * (_PR // _BR): (i + _p, 0)),
            pl.BlockSpec((H, 4 * D), _full),
            pl.BlockSpec((D, H), _full),
            pl.BlockSpec((D, H), _full),
            pl.BlockSpec((1, H), _full),
            pl.BlockSpec((1, H), _full),
            pl.BlockSpec((1, H), _full),
        ],
        out_specs=[
            pl.BlockSpec((_BR, H), lambda i: (i, 0)),
            pl.BlockSpec((2, H), _full),
        ],
        out_shape=[
            jax.ShapeDtypeStruct((_PR, H), jnp.float32),
            jax.ShapeDtypeStruct((2, H), jnp.float32),
        ],
    )(gx3, x_review, whhT, wself, wneigh, bsage, lng, lnb)


_BPP = _PR // _BR


def _stage_b_body(*refs):
    hn_refs = refs[:_P]
    st_refs = refs[_P:2 * _P]
    bng_ref, bnb_ref, w1_ref, b1_ref, w2_ref, b2_ref, out_ref = refs[2 * _P:]
    f32 = jnp.float32
    s = st_refs[0][...]
    for r in st_refs[1:]:
        s = s + r[...]
    bm = s[0:1, :] * (1.0 / N_REV)
    bv = s[1:2, :] * (1.0 / N_REV) - bm * bm
    scale = lax.rsqrt(bv + 1e-5) * bng_ref[...]
    p = pl.program_id(0) // _BPP
    hn = hn_refs[0][...]
    for q in range(1, _P):
        hn = jnp.where(p == q, hn_refs[q][...], hn)
    hb = (hn - bm) * scale + bnb_ref[...]
    t1 = jax.nn.gelu(jnp.dot(hb, w1_ref[...], preferred_element_type=f32)
                     + b1_ref[...])
    out_ref[...] = jnp.dot(t1, w2_ref[...], preferred_element_type=f32) + b2_ref[...]


def _stage_b(hns, stats_parts, bng, bnb, w1, b1, w2, b2):
    full = lambda i: (0, 0)

    def hn_map(q):
        return lambda i: (jnp.clip(i - q * _BPP, 0, _BPP - 1), 0)

    return pl.pallas_call(
        _stage_b_body,
        grid=(N_REV // _BR,),
        in_specs=[pl.BlockSpec((_BR, H), hn_map(q)) for q in range(_P)]
        + [pl.BlockSpec((2, H), full)] * _P
        + [
            pl.BlockSpec((1, H), full),
            pl.BlockSpec((1, H), full),
            pl.BlockSpec((H, H), full),
            pl.BlockSpec((1, H), full),
            pl.BlockSpec((H, 2), full),
            pl.BlockSpec((1, 2), full),
        ],
        out_specs=pl.BlockSpec((_BR, 2), lambda i: (i, 0)),
        out_shape=jax.ShapeDtypeStruct((N_REV, 2), jnp.float32),
    )(*hns, *stats_parts, bng, bnb, w1, b1, w2, b2)


def kernel(x_user, x_review, edge_src, W_ih, W_hh, b_ih, b_hh, W_self,
           W_neigh, b_sage, ln_g, ln_b, bn_g, bn_b, W1, b1, W2, b2):
    idx_t = jnp.transpose(
        edge_src.reshape(_P, _PR, DEG), (0, 2, 1)
    ).reshape(_P, _NW, _NCHUNK, _CHUNK)

    gate_scale = jnp.concatenate(
        [jnp.full((D,), 0.5, jnp.float32), jnp.full((D,), 0.5, jnp.float32),
         jnp.ones((D,), jnp.float32), jnp.full((D,), 0.5, jnp.float32)])
    wihT = (W_ih.T * gate_scale).astype(jnp.bfloat16)
    whhT = (W_hh.T * gate_scale).astype(jnp.bfloat16)
    b = ((b_ih + b_hh) * gate_scale).reshape(1, 4 * D)

    table = _stage_u(x_user, wihT, b)
    gx = [_gather_sc(p, idx_t, table).reshape(DEG, _PR, _GW)
          for p in range(_P)]

    hns, stats_parts = [], []
    for p in range(_P):
        hn_p, st_p = _stage_a_phase(p, gx[p], x_review, whhT, W_self,
                                    W_neigh, b_sage.reshape(1, H),
                                    ln_g.reshape(1, H), ln_b.reshape(1, H))
        hns.append(hn_p)
        stats_parts.append(st_p)

    return _stage_b(hns, stats_parts, bn_g.reshape(1, H), bn_b.reshape(1, H),
                    W1, b1.reshape(1, H), W2, b2.reshape(1, 2))
